# baseline (speedup 1.0000x reference)
import jax
import jax.numpy as jnp
from jax.experimental import pallas as pl

IN_CHANNELS = 256
HIDDEN_DIM = 128
NUM_EXPERTS = 4


def _gate_kernel(x0_ref, x1_ref, w1_ref, b1_ref, w2_ref, b2_ref, out_ref):
    b = pl.program_id(0)
    x0, x1 = x0_ref[0], x1_ref[0]
    hw = 2 * x0.shape[0] * x0.shape[1]
    part = jnp.sum(x0, axis=0) + jnp.sum(x1, axis=0)
    pooled = jnp.sum(part, axis=0) * (1.0 / hw)
    pooled = pooled.reshape(1, -1)
    h = jnp.dot(pooled, w1_ref[...], preferred_element_type=jnp.float32)
    h = h + b1_ref[...]
    h = 0.5 * h * (1.0 + jax.lax.erf(h * 0.7071067811865476))
    logits = jnp.dot(h, w2_ref[...], preferred_element_type=jnp.float32)
    logits = logits + b2_ref[...]
    m = jnp.max(logits, axis=-1, keepdims=True)
    e = jnp.exp(logits - m)
    weights = e / jnp.sum(e, axis=-1, keepdims=True)
    out_ref[pl.ds(b, 1), :] = weights


@jax.jit
def kernel(img_emb, W1, b1, W2, b2):
    B, C, H, W = img_emb.shape
    x = img_emb.transpose(0, 2, 3, 1)
    b1r = b1.reshape(1, HIDDEN_DIM)
    b2r = b2.reshape(1, NUM_EXPERTS)
    Hh = H // 2

    def hspec(k):
        return pl.BlockSpec((1, Hh, W, C), lambda b, k=k: (b, k, 0, 0))

    out = pl.pallas_call(
        _gate_kernel,
        grid=(B,),
        in_specs=[
            hspec(0), hspec(1),
            pl.BlockSpec((C, HIDDEN_DIM), lambda b: (0, 0)),
            pl.BlockSpec((1, HIDDEN_DIM), lambda b: (0, 0)),
            pl.BlockSpec((HIDDEN_DIM, NUM_EXPERTS), lambda b: (0, 0)),
            pl.BlockSpec((1, NUM_EXPERTS), lambda b: (0, 0)),
        ],
        out_specs=pl.BlockSpec((B, NUM_EXPERTS), lambda b: (0, 0)),
        out_shape=jax.ShapeDtypeStruct((B, NUM_EXPERTS), jnp.float32),
    )(x, x, W1, b1r, W2, b2r)
    return out


# speedup vs baseline: 1.1950x; 1.1950x over previous
import jax
import jax.numpy as jnp
from jax.experimental import pallas as pl

IN_CHANNELS = 256
HIDDEN_DIM = 128
NUM_EXPERTS = 4
BB = 2


def _gate_kernel(x_ref, w1_ref, b1_ref, w2_ref, b2_ref, out_ref):
    g = pl.program_id(0)
    hw = x_ref.shape[1] * x_ref.shape[2]
    rows = []
    for i in range(BB):
        x = x_ref[i]
        part = jnp.sum(x, axis=0)
        rows.append(jnp.sum(part, axis=0))
    pooled = jnp.stack(rows, axis=0) * (1.0 / hw)
    h = jnp.dot(pooled, w1_ref[...], preferred_element_type=jnp.float32)
    h = h + b1_ref[...]
    h = 0.5 * h * (1.0 + jax.lax.erf(h * 0.7071067811865476))
    logits = jnp.dot(h, w2_ref[...], preferred_element_type=jnp.float32)
    logits = logits + b2_ref[...]
    m = jnp.max(logits, axis=-1, keepdims=True)
    e = jnp.exp(logits - m)
    weights = e / jnp.sum(e, axis=-1, keepdims=True)
    out_ref[pl.ds(g * BB, BB), :] = weights


@jax.jit
def kernel(img_emb, W1, b1, W2, b2):
    B, C, H, W = img_emb.shape
    x = img_emb.transpose(0, 2, 3, 1)
    b1r = b1.reshape(1, HIDDEN_DIM)
    b2r = b2.reshape(1, NUM_EXPERTS)
    out = pl.pallas_call(
        _gate_kernel,
        grid=(B // BB,),
        in_specs=[
            pl.BlockSpec((BB, H, W, C), lambda g: (g, 0, 0, 0)),
            pl.BlockSpec((C, HIDDEN_DIM), lambda g: (0, 0)),
            pl.BlockSpec((1, HIDDEN_DIM), lambda g: (0, 0)),
            pl.BlockSpec((HIDDEN_DIM, NUM_EXPERTS), lambda g: (0, 0)),
            pl.BlockSpec((1, NUM_EXPERTS), lambda g: (0, 0)),
        ],
        out_specs=pl.BlockSpec((B, NUM_EXPERTS), lambda g: (0, 0)),
        out_shape=jax.ShapeDtypeStruct((B, NUM_EXPERTS), jnp.float32),
    )(x, W1, b1r, W2, b2r)
    return out
